# Initial kernel scaffold; baseline (speedup 1.0000x reference)
import jax
import jax.numpy as jnp
from jax import lax
from jax.experimental import pallas as pl
from jax.experimental.pallas import tpu as pltpu
from jax.experimental.pallas import tpu_sc as plsc

_C = 16
_D = 32
_N = _D * _D * _D
_K = 5
_R = 3
_ROWS = _N // 128
_NW = 32
_CHUNK = _N // _NW
_HALO = 1024
_WIN = _CHUNK + 2 * _HALO
_LEVELS = 2
_SCALES = (2.0 ** -11, 2.0 ** -12)


def _pool_body(ff_ref, fm_ref, fix_ref, mov_ref):
    i = pl.program_id(0)

    @pl.when(i == 0)
    def _():
        fix_ref[...] = jnp.zeros_like(fix_ref)
        mov_ref[...] = jnp.zeros_like(mov_ref)

    fix_ref[...] += ff_ref[0] * 0.125
    mov_ref[...] += fm_ref[0] * 0.125


_pool = pl.pallas_call(
    _pool_body,
    grid=(8,),
    in_specs=[
        pl.BlockSpec((1, _C, _ROWS, 128), lambda i: (i, 0, 0, 0)),
        pl.BlockSpec((1, _C, _ROWS, 128), lambda i: (i, 0, 0, 0)),
    ],
    out_specs=[
        pl.BlockSpec((_C, _ROWS, 128), lambda i: (0, 0, 0)),
        pl.BlockSpec((_C, _ROWS, 128), lambda i: (0, 0, 0)),
    ],
    out_shape=[
        jax.ShapeDtypeStruct((_C, _ROWS, 128), jnp.float32),
        jax.ShapeDtypeStruct((_C, _ROWS, 128), jnp.float32),
    ],
)


def _corr_body(fix_ref, mov_ref, vals_ref, oids_ref):
    fix = fix_ref[...]
    mov = mov_ref[...]
    movB = jnp.concatenate([mov[:, -32:, :], mov, mov[:, :32, :]], axis=1)

    row = lax.broadcasted_iota(jnp.int32, (_ROWS, 128), 0)
    lane = lax.broadcasted_iota(jnp.int32, (_ROWS, 128), 1)
    n = row * 128 + lane
    z = n >> 10
    y = (n >> 5) & 31
    x = n & 31

    keys = [jnp.zeros((_ROWS, 128), jnp.uint32) for _ in range(_K)]

    for dy in range(-_R, _R + 1):
        for dx in range(-_R, _R + 1):
            r = dy * 32 + dx
            if r > 0:
                rw = jnp.concatenate(
                    [movB[:, 8:312, r:], movB[:, 9:313, :r]], axis=2)
            elif r < 0:
                rw = jnp.concatenate(
                    [movB[:, 7:311, 128 + r:], movB[:, 8:312, :128 + r]],
                    axis=2)
            else:
                rw = movB[:, 8:312, :]

            vm = None
            if dy > 0:
                vm = y < 32 - dy
            elif dy < 0:
                vm = y >= -dy
            if dx > 0:
                vm = (x < 32 - dx) if vm is None else vm & (x < 32 - dx)
            elif dx < 0:
                vm = (x >= -dx) if vm is None else vm & (x >= -dx)

            for dz in range(-_R, _R + 1):
                w = rw[:, 24 + dz * 8:280 + dz * 8, :]
                c = jnp.sum(fix * w, axis=0) * 0.25
                m = vm
                if dz > 0:
                    m = (z < 32 - dz) if m is None else m & (z < 32 - dz)
                elif dz < 0:
                    m = (z >= -dz) if m is None else m & (z >= -dz)
                if m is not None:
                    c = jnp.where(m, c, -jnp.inf)
                o = (dz + _R) * 49 + (dy + _R) * 7 + (dx + _R)
                b = lax.bitcast_convert_type(c, jnp.uint32)
                mk = jnp.where(c >= 0, b | jnp.uint32(0x80000000), ~b)
                t = (mk & jnp.uint32(0xFFFFFE00)) | jnp.uint32(o)
                for j in range(_K):
                    nk = jnp.maximum(keys[j], t)
                    t = jnp.minimum(keys[j], t)
                    keys[j] = nk

    for k in range(_K):
        kk = keys[k]
        o = (kk & jnp.uint32(511)).astype(jnp.int32)
        mk = kk & jnp.uint32(0xFFFFFE00)
        neg = mk < jnp.uint32(0x80000000)
        bits = jnp.where(neg, ~mk, mk & jnp.uint32(0x7FFFFFFF))
        vals_ref[k] = lax.bitcast_convert_type(bits, jnp.float32)
        oids_ref[k] = o


_corr = pl.pallas_call(
    _corr_body,
    out_shape=[
        jax.ShapeDtypeStruct((_K, _ROWS, 128), jnp.float32),
        jax.ShapeDtypeStruct((_K, _ROWS, 128), jnp.int32),
    ],
)


def _sc_body(vals_hbm, oids_hbm, win_hbm, vals_v, oids_v, acc_v):
    wid = lax.axis_index("s") * 2 + lax.axis_index("c")
    base = wid * _CHUNK
    pltpu.sync_copy(vals_hbm.at[:, pl.ds(base, _CHUNK)], vals_v)
    pltpu.sync_copy(oids_hbm.at[:, pl.ds(base, _CHUNK)], oids_v)

    zeros16 = jnp.zeros((16,), jnp.float32)

    def _zero(i, carry):
        acc_v[pl.ds(i * 16, 16)] = zeros16
        return carry

    lax.fori_loop(0, (_LEVELS * _WIN) // 16, _zero, 0)

    lanes = lax.iota(jnp.int32, (16,))

    def _group(g, carry):
        nvec = base + g * 16 + lanes
        z = nvec >> 10
        yv = (nvec >> 5) & 31
        xv = nvec & 31
        loc = g * 16 + _HALO + lanes
        for k in range(_K):
            v = vals_v[k, pl.ds(g * 16, 16)]
            o = oids_v[k, pl.ds(g * 16, 16)]
            oz = (o * 1338) >> 16
            rem = o - oz * 49
            oy = (rem * 9363) >> 16
            ox = rem - oy * 7
            dz = oz - _R
            dy = oy - _R
            dx = ox - _R
            ez = jnp.clip(z + jnp.sign(dz), 0, 31) - z
            ey = jnp.clip(yv + jnp.sign(dy), 0, 31) - yv
            ex = jnp.clip(xv + jnp.sign(dx), 0, 31) - xv
            iz = ez << 10
            iy = ey << 5
            azf = jnp.abs(dz).astype(jnp.float32)
            ayf = jnp.abs(dy).astype(jnp.float32)
            axf = jnp.abs(dx).astype(jnp.float32)
            for l in range(_LEVELS):
                s = _SCALES[l]
                az = azf * s
                ay = ayf * s
                ax = axf * s
                wz0 = 1.0 - az
                wy0 = 1.0 - ay
                wx0 = 1.0 - ax
                b = loc + l * _WIN
                plsc.addupdate_scatter(acc_v, [b], ((wz0 * wy0) * wx0) * v)
                plsc.addupdate_scatter(acc_v, [b + iz], ((az * wy0) * wx0) * v)
                plsc.addupdate_scatter(acc_v, [b + iy], ((wz0 * ay) * wx0) * v)
                plsc.addupdate_scatter(acc_v, [b + ex], ((wz0 * wy0) * ax) * v)
                plsc.addupdate_scatter(
                    acc_v, [b + iz + iy], ((az * ay) * wx0) * v)
                plsc.addupdate_scatter(
                    acc_v, [b + iz + ex], ((az * wy0) * ax) * v)
                plsc.addupdate_scatter(
                    acc_v, [b + iy + ex], ((wz0 * ay) * ax) * v)
                plsc.addupdate_scatter(
                    acc_v, [b + iz + iy + ex], ((az * ay) * ax) * v)
        return carry

    lax.fori_loop(0, _CHUNK // 16, _group, 0)
    pltpu.sync_copy(acc_v, win_hbm.at[wid])


_scatter = pl.kernel(
    _sc_body,
    out_type=jax.ShapeDtypeStruct((_NW, _LEVELS * _WIN), jnp.float32),
    mesh=plsc.VectorSubcoreMesh(core_axis_name="c", subcore_axis_name="s"),
    scratch_types=[
        pltpu.VMEM((_K, _CHUNK), jnp.float32),
        pltpu.VMEM((_K, _CHUNK), jnp.int32),
        pltpu.VMEM((_LEVELS * _WIN,), jnp.float32),
    ],
)


def _merge_body(win_ref, out_ref):
    out_ref[...] = jnp.zeros_like(out_ref)
    for w in range(_NW):
        s = w * _CHUNK - _HALO
        lo = max(s, 0)
        hi = min(s + _WIN, _N)
        ln = hi - lo
        for l in range(_LEVELS):
            out_ref[l, pl.ds(lo, ln)] += win_ref[w, pl.ds(l * _WIN + lo - s, ln)]


_merge = pl.pallas_call(
    _merge_body,
    out_shape=jax.ShapeDtypeStruct((_LEVELS, _N), jnp.float32),
)


def _taps(feat):
    f = feat.reshape(_C, _D, 2, _D, 2, _D, 2)
    return f.transpose(2, 4, 6, 0, 1, 3, 5).reshape(8, _C, _ROWS, 128)


def kernel(feat_fix, feat_mov):
    fix, mov = _pool(_taps(feat_fix), _taps(feat_mov))
    vals, oids = _corr(fix, mov)
    win = _scatter(vals.reshape(_K, _N), oids.reshape(_K, _N))
    dense = _merge(win)
    return dense.reshape(_LEVELS, _D, _D, _D)


# speedup vs baseline: 44.9598x; 44.9598x over previous
import jax
import jax.numpy as jnp
from jax import lax
from jax.experimental import pallas as pl
from jax.experimental.pallas import tpu as pltpu
from jax.experimental.pallas import tpu_sc as plsc

_C = 16
_D = 32
_N = _D * _D * _D
_K = 5
_R = 3
_ROWS = _N // 128
_NW = 32
_CHUNK = _N // _NW
_HALO = 1024
_WIN = _CHUNK + 2 * _HALO
_LEVELS = 2
_SCALES = (2.0 ** -11, 2.0 ** -12)


def _pool_body(ff_ref, fm_ref, fix_ref, mov_ref):
    i = pl.program_id(0)

    @pl.when(i == 0)
    def _():
        fix_ref[...] = jnp.zeros_like(fix_ref)
        mov_ref[...] = jnp.zeros_like(mov_ref)

    fix_ref[...] += ff_ref[0] * 0.125
    mov_ref[...] += fm_ref[0] * 0.125


_pool = pl.pallas_call(
    _pool_body,
    grid=(8,),
    in_specs=[
        pl.BlockSpec((1, _C, _ROWS, 128), lambda i: (i, 0, 0, 0)),
        pl.BlockSpec((1, _C, _ROWS, 128), lambda i: (i, 0, 0, 0)),
    ],
    out_specs=[
        pl.BlockSpec((_C, _ROWS, 128), lambda i: (0, 0, 0)),
        pl.BlockSpec((_C, _ROWS, 128), lambda i: (0, 0, 0)),
    ],
    out_shape=[
        jax.ShapeDtypeStruct((_C, _ROWS, 128), jnp.float32),
        jax.ShapeDtypeStruct((_C, _ROWS, 128), jnp.float32),
    ],
)


def _corr_body(fix_ref, mov_ref, vals_ref, oids_ref, rw_ref):
    fix = fix_ref[...]
    mov = mov_ref[...]
    movB = jnp.concatenate([mov[:, -32:, :], mov, mov[:, :32, :]], axis=1)

    row = lax.broadcasted_iota(jnp.int32, (_ROWS, 128), 0)
    lane = lax.broadcasted_iota(jnp.int32, (_ROWS, 128), 1)
    n = row * 128 + lane
    z = n >> 10
    y = (n >> 5) & 31
    x = n & 31

    keys = [jnp.full((_ROWS, 128), -0x80000000, jnp.int32) for _ in range(_K)]

    for dy in range(-_R, _R + 1):
        for dx in range(-_R, _R + 1):
            r = dy * 32 + dx
            if r > 0:
                rw_ref[...] = jnp.concatenate(
                    [movB[:, 8:312, r:], movB[:, 9:313, :r]], axis=2)
            elif r < 0:
                rw_ref[...] = jnp.concatenate(
                    [movB[:, 7:311, 128 + r:], movB[:, 8:312, :128 + r]],
                    axis=2)
            else:
                rw_ref[...] = movB[:, 8:312, :]

            vm = None
            if dy > 0:
                vm = y < 32 - dy
            elif dy < 0:
                vm = y >= -dy
            if dx > 0:
                vm = (x < 32 - dx) if vm is None else vm & (x < 32 - dx)
            elif dx < 0:
                vm = (x >= -dx) if vm is None else vm & (x >= -dx)
            obase = (dy + _R) * 7 + (dx + _R)

            def _dz_body(dzi, ks, vm=vm, obase=obase):
                w = rw_ref[:, pl.ds(pl.multiple_of(8 * dzi, 8), _ROWS), :]
                c = jnp.sum(fix * w, axis=0) * 0.25
                dz = dzi - _R
                m = (z + dz >= 0) & (z + dz < 32)
                if vm is not None:
                    m = m & vm
                c = jnp.where(m, c, -jnp.inf)
                o = (dzi * 49 + obase).astype(jnp.uint32)
                b = lax.bitcast_convert_type(c, jnp.uint32)
                mk = jnp.where(c >= 0, b | jnp.uint32(0x80000000), ~b)
                mk = (mk & jnp.uint32(0xFFFFFE00)) | o
                t = lax.bitcast_convert_type(
                    mk ^ jnp.uint32(0x80000000), jnp.int32)
                ks = list(ks)
                for j in range(_K):
                    nk = jnp.maximum(ks[j], t)
                    t = jnp.minimum(ks[j], t)
                    ks[j] = nk
                return tuple(ks)

            keys = list(lax.fori_loop(0, 7, _dz_body, tuple(keys)))

    for k in range(_K):
        kk = lax.bitcast_convert_type(keys[k], jnp.uint32) ^ jnp.uint32(
            0x80000000)
        o = lax.bitcast_convert_type(kk & jnp.uint32(511), jnp.int32)
        mk = kk & jnp.uint32(0xFFFFFE00)
        neg = keys[k] < 0
        bits = jnp.where(neg, ~mk, mk & jnp.uint32(0x7FFFFFFF))
        vals_ref[k] = lax.bitcast_convert_type(bits, jnp.float32)
        oids_ref[k] = o


_corr = pl.pallas_call(
    _corr_body,
    out_shape=[
        jax.ShapeDtypeStruct((_K, _ROWS, 128), jnp.float32),
        jax.ShapeDtypeStruct((_K, _ROWS, 128), jnp.int32),
    ],
    scratch_shapes=[pltpu.VMEM((_C, 304, 128), jnp.float32)],
    compiler_params=pltpu.CompilerParams(vmem_limit_bytes=100 * 1024 * 1024),
)


def _sc_body(vals_hbm, oids_hbm, win_hbm, vals_v, oids_v, acc_v):
    wid = lax.axis_index("s") * 2 + lax.axis_index("c")
    base = wid * _CHUNK
    pltpu.sync_copy(vals_hbm.at[:, pl.ds(base, _CHUNK)], vals_v)
    pltpu.sync_copy(oids_hbm.at[:, pl.ds(base, _CHUNK)], oids_v)

    zeros16 = jnp.zeros((16,), jnp.float32)

    def _zero(i, carry):
        acc_v[pl.ds(i * 16, 16)] = zeros16
        return carry

    lax.fori_loop(0, (_LEVELS * _WIN) // 16, _zero, 0)

    lanes = lax.iota(jnp.int32, 16)

    def _group(g, carry):
        nvec = base + g * 16 + lanes
        z = nvec >> 10
        yv = (nvec >> 5) & 31
        xv = nvec & 31
        loc = g * 16 + _HALO + lanes
        for k in range(_K):
            v = vals_v[k, pl.ds(g * 16, 16)]
            o = oids_v[k, pl.ds(g * 16, 16)]
            oz = (o * 1338) >> 16
            rem = o - oz * 49
            oy = (rem * 9363) >> 16
            ox = rem - oy * 7
            dz = oz - _R
            dy = oy - _R
            dx = ox - _R
            ez = jnp.clip(z + jnp.sign(dz), 0, 31) - z
            ey = jnp.clip(yv + jnp.sign(dy), 0, 31) - yv
            ex = jnp.clip(xv + jnp.sign(dx), 0, 31) - xv
            iz = ez << 10
            iy = ey << 5
            azf = jnp.abs(dz).astype(jnp.float32)
            ayf = jnp.abs(dy).astype(jnp.float32)
            axf = jnp.abs(dx).astype(jnp.float32)
            for l in range(_LEVELS):
                s = _SCALES[l]
                az = azf * s
                ay = ayf * s
                ax = axf * s
                wz0 = 1.0 - az
                wy0 = 1.0 - ay
                wx0 = 1.0 - ax
                b = loc + l * _WIN
                plsc.addupdate_scatter(acc_v, [b], ((wz0 * wy0) * wx0) * v)
                plsc.addupdate_scatter(acc_v, [b + iz], ((az * wy0) * wx0) * v)
                plsc.addupdate_scatter(acc_v, [b + iy], ((wz0 * ay) * wx0) * v)
                plsc.addupdate_scatter(acc_v, [b + ex], ((wz0 * wy0) * ax) * v)
                plsc.addupdate_scatter(
                    acc_v, [b + iz + iy], ((az * ay) * wx0) * v)
                plsc.addupdate_scatter(
                    acc_v, [b + iz + ex], ((az * wy0) * ax) * v)
                plsc.addupdate_scatter(
                    acc_v, [b + iy + ex], ((wz0 * ay) * ax) * v)
                plsc.addupdate_scatter(
                    acc_v, [b + iz + iy + ex], ((az * ay) * ax) * v)
        return carry

    lax.fori_loop(0, _CHUNK // 16, _group, 0)
    pltpu.sync_copy(acc_v, win_hbm.at[wid])


_scatter_cache = []


def _scatter(vals, oids):
    if not _scatter_cache:
        _scatter_cache.append(pl.kernel(
            _sc_body,
            compiler_params=pltpu.CompilerParams(needs_layout_passes=False),
            out_type=jax.ShapeDtypeStruct((_NW, _LEVELS * _WIN), jnp.float32),
            mesh=plsc.VectorSubcoreMesh(
                core_axis_name="c", subcore_axis_name="s"),
            scratch_types=[
                pltpu.VMEM((_K, _CHUNK), jnp.float32),
                pltpu.VMEM((_K, _CHUNK), jnp.int32),
                pltpu.VMEM((_LEVELS * _WIN,), jnp.float32),
            ],
        ))
    return _scatter_cache[0](vals, oids)


def _merge_body(win_ref, out_ref):
    out_ref[...] = jnp.zeros_like(out_ref)
    for w in range(_NW):
        s = w * _CHUNK - _HALO
        lo = max(s, 0)
        hi = min(s + _WIN, _N)
        ln = hi - lo
        for l in range(_LEVELS):
            out_ref[l, pl.ds(lo, ln)] += win_ref[w, pl.ds(l * _WIN + lo - s, ln)]


_merge = pl.pallas_call(
    _merge_body,
    out_shape=jax.ShapeDtypeStruct((_LEVELS, _N), jnp.float32),
)


def _taps(feat):
    f = feat.reshape(_C, _D, 2, _D, 2, _D, 2)
    return f.transpose(2, 4, 6, 0, 1, 3, 5).reshape(8, _C, _ROWS, 128)


def kernel(feat_fix, feat_mov):
    fix, mov = _pool(_taps(feat_fix), _taps(feat_mov))
    vals, oids = _corr(fix, mov)
    win = _scatter(vals.reshape(_K, _N), oids.reshape(_K, _N))
    dense = _merge(win)
    return dense.reshape(_LEVELS, _D, _D, _D)


# speedup vs baseline: 45.0900x; 1.0029x over previous
import jax
import jax.numpy as jnp
from jax import lax
from jax.experimental import pallas as pl
from jax.experimental.pallas import tpu as pltpu
from jax.experimental.pallas import tpu_sc as plsc

_C = 16
_D = 32
_N = _D * _D * _D
_K = 5
_R = 3
_ROWS = _N // 128
_NW = 32
_CHUNK = _N // _NW
_HALO = 1024
_WIN = _CHUNK + 2 * _HALO
_LEVELS = 2
_SCALES = (2.0 ** -11, 2.0 ** -12)


def _pool_body(ff_ref, fm_ref, fix_ref, mov_ref):
    i = pl.program_id(0)

    @pl.when(i == 0)
    def _():
        fix_ref[...] = jnp.zeros_like(fix_ref)
        mov_ref[...] = jnp.zeros_like(mov_ref)

    fix_ref[...] += ff_ref[0] * 0.125
    mov_ref[...] += fm_ref[0] * 0.125


_pool = pl.pallas_call(
    _pool_body,
    grid=(8,),
    in_specs=[
        pl.BlockSpec((1, _C, _ROWS, 128), lambda i: (i, 0, 0, 0)),
        pl.BlockSpec((1, _C, _ROWS, 128), lambda i: (i, 0, 0, 0)),
    ],
    out_specs=[
        pl.BlockSpec((_C, _ROWS, 128), lambda i: (0, 0, 0)),
        pl.BlockSpec((_C, _ROWS, 128), lambda i: (0, 0, 0)),
    ],
    out_shape=[
        jax.ShapeDtypeStruct((_C, _ROWS, 128), jnp.float32),
        jax.ShapeDtypeStruct((_C, _ROWS, 128), jnp.float32),
    ],
)


def _corr_body(fix_ref, mov_ref, vals_ref, oids_ref, rw_ref):
    fix = fix_ref[...]
    mov = mov_ref[...]
    movB = jnp.concatenate([mov[:, -32:, :], mov, mov[:, :32, :]], axis=1)

    row = lax.broadcasted_iota(jnp.int32, (_ROWS, 128), 0)
    lane = lax.broadcasted_iota(jnp.int32, (_ROWS, 128), 1)
    n = row * 128 + lane
    z = n >> 10
    y = (n >> 5) & 31
    x = n & 31

    keys = [jnp.full((_ROWS, 128), -0x80000000, jnp.int32) for _ in range(_K)]

    for dy in range(-_R, _R + 1):
        for dx in range(-_R, _R + 1):
            r = dy * 32 + dx
            if r > 0:
                rw_ref[...] = jnp.concatenate(
                    [movB[:, 8:312, r:], movB[:, 9:313, :r]], axis=2)
            elif r < 0:
                rw_ref[...] = jnp.concatenate(
                    [movB[:, 7:311, 128 + r:], movB[:, 8:312, :128 + r]],
                    axis=2)
            else:
                rw_ref[...] = movB[:, 8:312, :]

            vm = None
            if dy > 0:
                vm = y < 32 - dy
            elif dy < 0:
                vm = y >= -dy
            if dx > 0:
                vm = (x < 32 - dx) if vm is None else vm & (x < 32 - dx)
            elif dx < 0:
                vm = (x >= -dx) if vm is None else vm & (x >= -dx)
            obase = (dy + _R) * 7 + (dx + _R)

            def _dz_body(dzi, ks, vm=vm, obase=obase):
                w = rw_ref[:, pl.ds(pl.multiple_of(8 * dzi, 8), _ROWS), :]
                c = jnp.sum(fix * w, axis=0) * 0.25
                dz = dzi - _R
                m = (z + dz >= 0) & (z + dz < 32)
                if vm is not None:
                    m = m & vm
                c = jnp.where(m, c, -jnp.inf)
                o = (dzi * 49 + obase).astype(jnp.uint32)
                b = lax.bitcast_convert_type(c, jnp.uint32)
                mk = jnp.where(c >= 0, b | jnp.uint32(0x80000000), ~b)
                mk = (mk & jnp.uint32(0xFFFFFE00)) | o
                t = lax.bitcast_convert_type(
                    mk ^ jnp.uint32(0x80000000), jnp.int32)
                ks = list(ks)
                for j in range(_K):
                    nk = jnp.maximum(ks[j], t)
                    t = jnp.minimum(ks[j], t)
                    ks[j] = nk
                return tuple(ks)

            keys = list(lax.fori_loop(0, 7, _dz_body, tuple(keys)))

    for k in range(_K):
        kk = lax.bitcast_convert_type(keys[k], jnp.uint32) ^ jnp.uint32(
            0x80000000)
        o = lax.bitcast_convert_type(kk & jnp.uint32(511), jnp.int32)
        mk = kk & jnp.uint32(0xFFFFFE00)
        neg = keys[k] < 0
        bits = jnp.where(neg, ~mk, mk & jnp.uint32(0x7FFFFFFF))
        vals_ref[k] = lax.bitcast_convert_type(bits, jnp.float32)
        oids_ref[k] = o


_corr = pl.pallas_call(
    _corr_body,
    out_shape=[
        jax.ShapeDtypeStruct((_K, _ROWS, 128), jnp.float32),
        jax.ShapeDtypeStruct((_K, _ROWS, 128), jnp.int32),
    ],
    scratch_shapes=[pltpu.VMEM((_C, 304, 128), jnp.float32)],
    compiler_params=pltpu.CompilerParams(vmem_limit_bytes=100 * 1024 * 1024),
)


def _sc_body(vals_hbm, oids_hbm, win_hbm, vals_v, oids_v, acc_v):
    wid = lax.axis_index("s") * 2 + lax.axis_index("c")
    base = wid * _CHUNK
    for k in range(_K):
        pltpu.sync_copy(vals_hbm.at[pl.ds(k * _N + base, _CHUNK)],
                        vals_v.at[pl.ds(k * _CHUNK, _CHUNK)])
        pltpu.sync_copy(oids_hbm.at[pl.ds(k * _N + base, _CHUNK)],
                        oids_v.at[pl.ds(k * _CHUNK, _CHUNK)])

    zeros16 = jnp.zeros((16,), jnp.float32)

    def _zero(i, carry):
        acc_v[pl.ds(i * 16, 16)] = zeros16
        return carry

    lax.fori_loop(0, (_LEVELS * _WIN) // 16, _zero, 0)

    lanes = lax.iota(jnp.int32, 16)

    def _group(g, carry):
        nvec = base + g * 16 + lanes
        z = nvec >> 10
        yv = (nvec >> 5) & 31
        xv = nvec & 31
        loc = g * 16 + _HALO + lanes
        for k in range(_K):
            v = vals_v[pl.ds(k * _CHUNK + g * 16, 16)]
            o = oids_v[pl.ds(k * _CHUNK + g * 16, 16)]
            oz = (o * 1338) >> 16
            rem = o - oz * 49
            oy = (rem * 9363) >> 16
            ox = rem - oy * 7
            dz = oz - _R
            dy = oy - _R
            dx = ox - _R
            ez = jnp.clip(z + jnp.sign(dz), 0, 31) - z
            ey = jnp.clip(yv + jnp.sign(dy), 0, 31) - yv
            ex = jnp.clip(xv + jnp.sign(dx), 0, 31) - xv
            iz = ez << 10
            iy = ey << 5
            azf = jnp.abs(dz).astype(jnp.float32)
            ayf = jnp.abs(dy).astype(jnp.float32)
            axf = jnp.abs(dx).astype(jnp.float32)
            for l in range(_LEVELS):
                s = _SCALES[l]
                az = azf * s
                ay = ayf * s
                ax = axf * s
                wz0 = 1.0 - az
                wy0 = 1.0 - ay
                wx0 = 1.0 - ax
                b = loc + l * _WIN
                plsc.addupdate_scatter(acc_v, [b], ((wz0 * wy0) * wx0) * v)
                plsc.addupdate_scatter(acc_v, [b + iz], ((az * wy0) * wx0) * v)
                plsc.addupdate_scatter(acc_v, [b + iy], ((wz0 * ay) * wx0) * v)
                plsc.addupdate_scatter(acc_v, [b + ex], ((wz0 * wy0) * ax) * v)
                plsc.addupdate_scatter(
                    acc_v, [b + iz + iy], ((az * ay) * wx0) * v)
                plsc.addupdate_scatter(
                    acc_v, [b + iz + ex], ((az * wy0) * ax) * v)
                plsc.addupdate_scatter(
                    acc_v, [b + iy + ex], ((wz0 * ay) * ax) * v)
                plsc.addupdate_scatter(
                    acc_v, [b + iz + iy + ex], ((az * ay) * ax) * v)
        return carry

    lax.fori_loop(0, _CHUNK // 16, _group, 0)
    pltpu.sync_copy(acc_v, win_hbm.at[pl.ds(wid * _LEVELS * _WIN, _LEVELS * _WIN)])


_scatter_cache = []


def _scatter(vals, oids):
    if not _scatter_cache:
        _scatter_cache.append(pl.kernel(
            _sc_body,
            compiler_params=pltpu.CompilerParams(
                needs_layout_passes=False, use_tc_tiling_on_sc=True),
            out_type=jax.ShapeDtypeStruct((_NW * _LEVELS * _WIN,), jnp.float32),
            mesh=plsc.VectorSubcoreMesh(
                core_axis_name="c", subcore_axis_name="s"),
            scratch_types=[
                pltpu.VMEM((_K * _CHUNK,), jnp.float32),
                pltpu.VMEM((_K * _CHUNK,), jnp.int32),
                pltpu.VMEM((_LEVELS * _WIN,), jnp.float32),
            ],
        ))
    return _scatter_cache[0](vals, oids)


def _merge_body(win_ref, out_ref):
    out_ref[...] = jnp.zeros_like(out_ref)
    for w in range(_NW):
        s = w * _CHUNK - _HALO
        lo = max(s, 0)
        hi = min(s + _WIN, _N)
        ln = hi - lo
        for l in range(_LEVELS):
            out_ref[l, pl.ds(lo, ln)] += win_ref[
                pl.ds(w * _LEVELS * _WIN + l * _WIN + lo - s, ln)]


_merge = pl.pallas_call(
    _merge_body,
    out_shape=jax.ShapeDtypeStruct((_LEVELS, _N), jnp.float32),
)


def _taps(feat):
    f = feat.reshape(_C, _D, 2, _D, 2, _D, 2)
    return f.transpose(2, 4, 6, 0, 1, 3, 5).reshape(8, _C, _ROWS, 128)


def kernel(feat_fix, feat_mov):
    fix, mov = _pool(_taps(feat_fix), _taps(feat_mov))
    vals, oids = _corr(fix, mov)
    win = _scatter(vals.reshape(_K * _N), oids.reshape(_K * _N))
    dense = _merge(win)
    return dense.reshape(_LEVELS, _D, _D, _D)


# speedup vs baseline: 45.4096x; 1.0071x over previous
import jax
import jax.numpy as jnp
from jax import lax
from jax.experimental import pallas as pl
from jax.experimental.pallas import tpu as pltpu
from jax.experimental.pallas import tpu_sc as plsc

_C = 16
_D = 32
_N = _D * _D * _D
_K = 5
_R = 3
_ROWS = _N // 128
_NW = 32
_CHUNK = _N // _NW
_HALO = 1024
_WIN = _CHUNK + 2 * _HALO
_LEVELS = 2
_SCALES = (2.0 ** -11, 2.0 ** -12)


def _pool_body(ff_ref, fm_ref, fix_ref, mov_ref):
    i = pl.program_id(0)

    @pl.when(i == 0)
    def _():
        fix_ref[...] = jnp.zeros_like(fix_ref)
        mov_ref[...] = jnp.zeros_like(mov_ref)

    fix_ref[...] += ff_ref[0] * 0.125
    mov_ref[...] += fm_ref[0] * 0.125


_pool = pl.pallas_call(
    _pool_body,
    grid=(8,),
    in_specs=[
        pl.BlockSpec((1, _C, _ROWS, 128), lambda i: (i, 0, 0, 0)),
        pl.BlockSpec((1, _C, _ROWS, 128), lambda i: (i, 0, 0, 0)),
    ],
    out_specs=[
        pl.BlockSpec((_C, _ROWS, 128), lambda i: (0, 0, 0)),
        pl.BlockSpec((_C, _ROWS, 128), lambda i: (0, 0, 0)),
    ],
    out_shape=[
        jax.ShapeDtypeStruct((_C, _ROWS, 128), jnp.float32),
        jax.ShapeDtypeStruct((_C, _ROWS, 128), jnp.float32),
    ],
)


def _corr_body(fix_ref, mov_ref, keys_ref, rw_ref):
    fix = fix_ref[...]
    mov = mov_ref[...]
    movB = jnp.concatenate([mov[:, -32:, :], mov, mov[:, :32, :]], axis=1)

    row = lax.broadcasted_iota(jnp.int32, (_ROWS, 128), 0)
    lane = lax.broadcasted_iota(jnp.int32, (_ROWS, 128), 1)
    n = row * 128 + lane
    z = n >> 10
    y = (n >> 5) & 31
    x = n & 31

    keys = [jnp.full((_ROWS, 128), -0x80000000, jnp.int32) for _ in range(_K)]

    for dy in range(-_R, _R + 1):
        for dx in range(-_R, _R + 1):
            r = dy * 32 + dx
            if r > 0:
                rw_ref[...] = jnp.concatenate(
                    [movB[:, 8:312, r:], movB[:, 9:313, :r]], axis=2)
            elif r < 0:
                rw_ref[...] = jnp.concatenate(
                    [movB[:, 7:311, 128 + r:], movB[:, 8:312, :128 + r]],
                    axis=2)
            else:
                rw_ref[...] = movB[:, 8:312, :]

            vm = None
            if dy > 0:
                vm = y < 32 - dy
            elif dy < 0:
                vm = y >= -dy
            if dx > 0:
                vm = (x < 32 - dx) if vm is None else vm & (x < 32 - dx)
            elif dx < 0:
                vm = (x >= -dx) if vm is None else vm & (x >= -dx)
            obase = (dy + _R) * 7 + (dx + _R)

            def _dz_body(dzi, ks, vm=vm, obase=obase):
                w = rw_ref[:, pl.ds(pl.multiple_of(8 * dzi, 8), _ROWS), :]
                c = jnp.sum(fix * w, axis=0) * 0.25
                dz = dzi - _R
                m = (z + dz >= 0) & (z + dz < 32)
                if vm is not None:
                    m = m & vm
                c = jnp.where(m, c, -jnp.inf)
                o = (dzi * 49 + obase).astype(jnp.uint32)
                b = lax.bitcast_convert_type(c, jnp.uint32)
                mk = jnp.where(c >= 0, b | jnp.uint32(0x80000000), ~b)
                mk = (mk & jnp.uint32(0xFFFFFE00)) | o
                t = lax.bitcast_convert_type(
                    mk ^ jnp.uint32(0x80000000), jnp.int32)
                ks = list(ks)
                for j in range(_K):
                    nk = jnp.maximum(ks[j], t)
                    t = jnp.minimum(ks[j], t)
                    ks[j] = nk
                return tuple(ks)

            keys = list(lax.fori_loop(0, 7, _dz_body, tuple(keys)))

    for k in range(_K):
        keys_ref[k] = keys[k]


_corr = pl.pallas_call(
    _corr_body,
    out_shape=jax.ShapeDtypeStruct((_K, _ROWS, 128), jnp.int32),
    scratch_shapes=[pltpu.VMEM((_C, 304, 128), jnp.float32)],
    compiler_params=pltpu.CompilerParams(vmem_limit_bytes=100 * 1024 * 1024),
)


def _sc_body(keys_hbm, win_hbm, keys_v, acc_v):
    wid = lax.axis_index("s") * 2 + lax.axis_index("c")
    base = wid * _CHUNK
    for k in range(_K):
        pltpu.sync_copy(keys_hbm.at[pl.ds(k * _N + base, _CHUNK)],
                        keys_v.at[pl.ds(k * _CHUNK, _CHUNK)])

    zeros16 = jnp.zeros((16,), jnp.float32)

    def _zero(i, carry):
        acc_v[pl.ds(i * 16, 16)] = zeros16
        return carry

    lax.fori_loop(0, (_LEVELS * _WIN) // 16, _zero, 0)

    lanes = lax.iota(jnp.int32, 16)

    def _group(g, carry):
        nvec = base + g * 16 + lanes
        z = nvec >> 10
        yv = (nvec >> 5) & 31
        xv = nvec & 31
        loc = g * 16 + _HALO + lanes
        for k in range(_K):
            ki = keys_v[pl.ds(k * _CHUNK + g * 16, 16)]
            ku = lax.bitcast_convert_type(ki, jnp.uint32) ^ jnp.uint32(
                0x80000000)
            o = lax.bitcast_convert_type(ku & jnp.uint32(511), jnp.int32)
            mku = ku & jnp.uint32(0xFFFFFE00)
            bits = jnp.where(ki < 0, ~mku, mku & jnp.uint32(0x7FFFFFFF))
            v = lax.bitcast_convert_type(bits, jnp.float32)
            oz = (o * 1338) >> 16
            rem = o - oz * 49
            oy = (rem * 9363) >> 16
            ox = rem - oy * 7
            dz = oz - _R
            dy = oy - _R
            dx = ox - _R
            ez = jnp.clip(z + jnp.sign(dz), 0, 31) - z
            ey = jnp.clip(yv + jnp.sign(dy), 0, 31) - yv
            ex = jnp.clip(xv + jnp.sign(dx), 0, 31) - xv
            iz = ez << 10
            iy = ey << 5
            azf = jnp.abs(dz).astype(jnp.float32)
            ayf = jnp.abs(dy).astype(jnp.float32)
            axf = jnp.abs(dx).astype(jnp.float32)
            for l in range(_LEVELS):
                s = _SCALES[l]
                az = azf * s
                ay = ayf * s
                ax = axf * s
                wz0 = 1.0 - az
                wy0 = 1.0 - ay
                wx0 = 1.0 - ax
                b = loc + l * _WIN
                plsc.addupdate_scatter(acc_v, [b], ((wz0 * wy0) * wx0) * v)
                plsc.addupdate_scatter(acc_v, [b + iz], ((az * wy0) * wx0) * v)
                plsc.addupdate_scatter(acc_v, [b + iy], ((wz0 * ay) * wx0) * v)
                plsc.addupdate_scatter(acc_v, [b + ex], ((wz0 * wy0) * ax) * v)
                plsc.addupdate_scatter(
                    acc_v, [b + iz + iy], ((az * ay) * wx0) * v)
                plsc.addupdate_scatter(
                    acc_v, [b + iz + ex], ((az * wy0) * ax) * v)
                plsc.addupdate_scatter(
                    acc_v, [b + iy + ex], ((wz0 * ay) * ax) * v)
                plsc.addupdate_scatter(
                    acc_v, [b + iz + iy + ex], ((az * ay) * ax) * v)
        return carry

    lax.fori_loop(0, _CHUNK // 16, _group, 0)
    pltpu.sync_copy(acc_v, win_hbm.at[pl.ds(wid * _LEVELS * _WIN, _LEVELS * _WIN)])


_scatter_cache = []


def _scatter(keys):
    if not _scatter_cache:
        _scatter_cache.append(pl.kernel(
            _sc_body,
            compiler_params=pltpu.CompilerParams(
                needs_layout_passes=False, use_tc_tiling_on_sc=True),
            out_type=jax.ShapeDtypeStruct((_NW * _LEVELS * _WIN,), jnp.float32),
            mesh=plsc.VectorSubcoreMesh(
                core_axis_name="c", subcore_axis_name="s"),
            scratch_types=[
                pltpu.VMEM((_K * _CHUNK,), jnp.int32),
                pltpu.VMEM((_LEVELS * _WIN,), jnp.float32),
            ],
        ))
    return _scatter_cache[0](keys)


def _merge_body(win_ref, out_ref):
    out_ref[...] = jnp.zeros_like(out_ref)
    for w in range(_NW):
        s = w * _CHUNK - _HALO
        lo = max(s, 0)
        hi = min(s + _WIN, _N)
        ln = hi - lo
        for l in range(_LEVELS):
            out_ref[l, pl.ds(lo, ln)] += win_ref[
                pl.ds(w * _LEVELS * _WIN + l * _WIN + lo - s, ln)]


_merge = pl.pallas_call(
    _merge_body,
    out_shape=jax.ShapeDtypeStruct((_LEVELS, _N), jnp.float32),
)


def _taps(feat):
    f = feat.reshape(_C, _D, 2, _D, 2, _D, 2)
    return f.transpose(2, 4, 6, 0, 1, 3, 5).reshape(8, _C, _ROWS, 128)


def kernel(feat_fix, feat_mov):
    fix, mov = _pool(_taps(feat_fix), _taps(feat_mov))
    keys = _corr(fix, mov)
    win = _scatter(keys.reshape(_K * _N))
    dense = _merge(win)
    return dense.reshape(_LEVELS, _D, _D, _D)


# speedup vs baseline: 57.9257x; 1.2756x over previous
import jax
import jax.numpy as jnp
from jax import lax
from jax.experimental import pallas as pl
from jax.experimental.pallas import tpu as pltpu
from jax.experimental.pallas import tpu_sc as plsc

_C = 16
_D = 32
_N = _D * _D * _D
_K = 5
_R = 3
_ROWS = _N // 128
_NW = 32
_CHUNK = _N // _NW
_HALO = 1024
_WIN = _CHUNK + 2 * _HALO
_LEVELS = 2
_SCALES = (2.0 ** -11, 2.0 ** -12)


def _pool_body(ff_ref, fm_ref, fix_ref, mov_ref):
    li = lax.broadcasted_iota(jnp.int32, (128, 32), 0)
    xi = lax.broadcasted_iota(jnp.int32, (128, 32), 1)
    p2 = (((li & 63) >> 1) == xi).astype(jnp.float32)
    dn = (((1,), (0,)), ((), ()))
    for src_ref, dst_ref in ((ff_ref, fix_ref), (fm_ref, mov_ref)):
        c1 = src_ref[:, :, 0] + src_ref[:, :, 1]
        c1 = c1.reshape(_C * 32 * 32, 128)
        d = lax.dot_general(c1, p2, dn,
                            preferred_element_type=jnp.float32) * 0.125
        dst_ref[...] = d.reshape(_C, 1024, 32)


_pool = pl.pallas_call(
    _pool_body,
    out_shape=[
        jax.ShapeDtypeStruct((_C, 1024, 32), jnp.float32),
        jax.ShapeDtypeStruct((_C, 1024, 32), jnp.float32),
    ],
    compiler_params=pltpu.CompilerParams(vmem_limit_bytes=63 * 1024 * 1024),
)


def _corr_body(fix_ref, mov_ref, keys_ref, rw_ref):
    fix = fix_ref[...]
    mov = mov_ref[...]
    movB = jnp.concatenate([mov[:, -32:, :], mov, mov[:, :32, :]], axis=1)

    row = lax.broadcasted_iota(jnp.int32, (_ROWS, 128), 0)
    lane = lax.broadcasted_iota(jnp.int32, (_ROWS, 128), 1)
    n = row * 128 + lane
    z = n >> 10
    y = (n >> 5) & 31
    x = n & 31

    keys = [jnp.full((_ROWS, 128), -0x80000000, jnp.int32) for _ in range(_K)]

    for dy in range(-_R, _R + 1):
        for dx in range(-_R, _R + 1):
            r = dy * 32 + dx
            if r > 0:
                rw_ref[...] = jnp.concatenate(
                    [movB[:, 8:312, r:], movB[:, 9:313, :r]], axis=2)
            elif r < 0:
                rw_ref[...] = jnp.concatenate(
                    [movB[:, 7:311, 128 + r:], movB[:, 8:312, :128 + r]],
                    axis=2)
            else:
                rw_ref[...] = movB[:, 8:312, :]

            vm = None
            if dy > 0:
                vm = y < 32 - dy
            elif dy < 0:
                vm = y >= -dy
            if dx > 0:
                vm = (x < 32 - dx) if vm is None else vm & (x < 32 - dx)
            elif dx < 0:
                vm = (x >= -dx) if vm is None else vm & (x >= -dx)
            obase = (dy + _R) * 7 + (dx + _R)

            def _dz_body(dzi, ks, vm=vm, obase=obase):
                w = rw_ref[:, pl.ds(pl.multiple_of(8 * dzi, 8), _ROWS), :]
                c = jnp.sum(fix * w, axis=0) * 0.25
                dz = dzi - _R
                m = (z + dz >= 0) & (z + dz < 32)
                if vm is not None:
                    m = m & vm
                c = jnp.where(m, c, -jnp.inf)
                o = (dzi * 49 + obase).astype(jnp.uint32)
                b = lax.bitcast_convert_type(c, jnp.uint32)
                mk = jnp.where(c >= 0, b | jnp.uint32(0x80000000), ~b)
                mk = (mk & jnp.uint32(0xFFFFFE00)) | o
                t = lax.bitcast_convert_type(
                    mk ^ jnp.uint32(0x80000000), jnp.int32)
                ks = list(ks)
                for j in range(_K):
                    nk = jnp.maximum(ks[j], t)
                    t = jnp.minimum(ks[j], t)
                    ks[j] = nk
                return tuple(ks)

            keys = list(lax.fori_loop(0, 7, _dz_body, tuple(keys)))

    for k in range(_K):
        keys_ref[k] = keys[k]


_corr = pl.pallas_call(
    _corr_body,
    out_shape=jax.ShapeDtypeStruct((_K, _ROWS, 128), jnp.int32),
    scratch_shapes=[pltpu.VMEM((_C, 304, 128), jnp.float32)],
    compiler_params=pltpu.CompilerParams(vmem_limit_bytes=63 * 1024 * 1024),
)


def _sc_body(keys_hbm, win_hbm, keys_v, acc_v):
    wid = lax.axis_index("s") * 2 + lax.axis_index("c")
    base = wid * _CHUNK
    for k in range(_K):
        pltpu.sync_copy(keys_hbm.at[pl.ds(k * _N + base, _CHUNK)],
                        keys_v.at[pl.ds(k * _CHUNK, _CHUNK)])

    zeros16 = jnp.zeros((16,), jnp.float32)

    def _zero(i, carry):
        acc_v[pl.ds(i * 16, 16)] = zeros16
        return carry

    lax.fori_loop(0, (_LEVELS * _WIN) // 16, _zero, 0)

    lanes = lax.iota(jnp.int32, 16)

    def _group(g, carry):
        nvec = base + g * 16 + lanes
        z = nvec >> 10
        yv = (nvec >> 5) & 31
        xv = nvec & 31
        loc = g * 16 + _HALO + lanes
        for k in range(_K):
            ki = keys_v[pl.ds(k * _CHUNK + g * 16, 16)]
            ku = lax.bitcast_convert_type(ki, jnp.uint32) ^ jnp.uint32(
                0x80000000)
            o = lax.bitcast_convert_type(ku & jnp.uint32(511), jnp.int32)
            mku = ku & jnp.uint32(0xFFFFFE00)
            bits = jnp.where(ki < 0, ~mku, mku & jnp.uint32(0x7FFFFFFF))
            v = lax.bitcast_convert_type(bits, jnp.float32)
            oz = (o * 1338) >> 16
            rem = o - oz * 49
            oy = (rem * 9363) >> 16
            ox = rem - oy * 7
            dz = oz - _R
            dy = oy - _R
            dx = ox - _R
            ez = jnp.clip(z + jnp.sign(dz), 0, 31) - z
            ey = jnp.clip(yv + jnp.sign(dy), 0, 31) - yv
            ex = jnp.clip(xv + jnp.sign(dx), 0, 31) - xv
            iz = ez << 10
            iy = ey << 5
            azf = jnp.abs(dz).astype(jnp.float32)
            ayf = jnp.abs(dy).astype(jnp.float32)
            axf = jnp.abs(dx).astype(jnp.float32)
            for l in range(_LEVELS):
                s = _SCALES[l]
                az = azf * s
                ay = ayf * s
                ax = axf * s
                wz0 = 1.0 - az
                wy0 = 1.0 - ay
                wx0 = 1.0 - ax
                b = loc + l * _WIN
                plsc.addupdate_scatter(acc_v, [b], ((wz0 * wy0) * wx0) * v)
                plsc.addupdate_scatter(acc_v, [b + iz], ((az * wy0) * wx0) * v)
                plsc.addupdate_scatter(acc_v, [b + iy], ((wz0 * ay) * wx0) * v)
                plsc.addupdate_scatter(acc_v, [b + ex], ((wz0 * wy0) * ax) * v)
                plsc.addupdate_scatter(
                    acc_v, [b + iz + iy], ((az * ay) * wx0) * v)
                plsc.addupdate_scatter(
                    acc_v, [b + iz + ex], ((az * wy0) * ax) * v)
                plsc.addupdate_scatter(
                    acc_v, [b + iy + ex], ((wz0 * ay) * ax) * v)
                plsc.addupdate_scatter(
                    acc_v, [b + iz + iy + ex], ((az * ay) * ax) * v)
        return carry

    lax.fori_loop(0, _CHUNK // 16, _group, 0)
    pltpu.sync_copy(acc_v, win_hbm.at[pl.ds(wid * _LEVELS * _WIN, _LEVELS * _WIN)])


_scatter_cache = []


def _scatter(keys):
    if not _scatter_cache:
        _scatter_cache.append(pl.kernel(
            _sc_body,
            compiler_params=pltpu.CompilerParams(
                needs_layout_passes=False, use_tc_tiling_on_sc=True),
            out_type=jax.ShapeDtypeStruct((_NW * _LEVELS * _WIN,), jnp.float32),
            mesh=plsc.VectorSubcoreMesh(
                core_axis_name="c", subcore_axis_name="s"),
            scratch_types=[
                pltpu.VMEM((_K * _CHUNK,), jnp.int32),
                pltpu.VMEM((_LEVELS * _WIN,), jnp.float32),
            ],
        ))
    return _scatter_cache[0](keys)


def _merge_body(win_ref, out_ref):
    out_ref[...] = jnp.zeros_like(out_ref)
    for w in range(_NW):
        s = w * _CHUNK - _HALO
        lo = max(s, 0)
        hi = min(s + _WIN, _N)
        ln = hi - lo
        for l in range(_LEVELS):
            out_ref[l, pl.ds(lo, ln)] += win_ref[
                pl.ds(w * _LEVELS * _WIN + l * _WIN + lo - s, ln)]


_merge = pl.pallas_call(
    _merge_body,
    out_shape=jax.ShapeDtypeStruct((_LEVELS, _N), jnp.float32),
)


def kernel(feat_fix, feat_mov):
    fix, mov = _pool(feat_fix.reshape(_C, 32, 2, 32, 128),
                     feat_mov.reshape(_C, 32, 2, 32, 128))
    keys = _corr(fix.reshape(_C, _ROWS, 128), mov.reshape(_C, _ROWS, 128))
    win = _scatter(keys.reshape(_K * _N))
    dense = _merge(win)
    return dense.reshape(_LEVELS, _D, _D, _D)
